# Initial kernel scaffold; baseline (speedup 1.0000x reference)
import jax
import jax.numpy as jnp
from jax.experimental import pallas as pl


def kernel(x, emb_table):
    raise NotImplementedError("write your pallas kernel here")


# speedup vs baseline: 1.4675x; 1.4675x over previous
import functools
import math

import jax
import jax.numpy as jnp
from jax import lax
from jax.experimental import pallas as pl
from jax.experimental.pallas import tpu as pltpu
from jax.experimental.pallas import tpu_sc as plsc

D_MODEL = 768
MAX_LEN = 8192
LANES = 16
SCALE = math.sqrt(D_MODEL)


def _positional_table():
    pos = jnp.arange(MAX_LEN, dtype=jnp.float32)[:, None]
    div_term = jnp.exp(
        jnp.arange(0, D_MODEL, 2, dtype=jnp.float32)
        * (-(math.log(10000.0) / D_MODEL))
    )
    pe = jnp.zeros((MAX_LEN, D_MODEL), dtype=jnp.float32)
    pe = pe.at[:, 0::2].set(jnp.sin(pos * div_term))
    pe = pe.at[:, 1::2].set(jnp.cos(pos * div_term))
    return pe


@functools.partial(jax.jit, static_argnames=("batch", "seq_len"))
def _embed(x_flat, pe, emb_table, *, batch, seq_len):
    n_tok = batch * seq_len
    info = plsc.get_sparse_core_info()
    nc, ns = info.num_cores, info.num_subcores
    nw = nc * ns
    bpw = n_tok // nw
    chunk = 64
    n_chunks = bpw // chunk
    vecs_per_row = D_MODEL // LANES
    mesh = plsc.VectorSubcoreMesh(core_axis_name="c", subcore_axis_name="s")

    @functools.partial(
        pl.kernel,
        mesh=mesh,
        out_type=jax.ShapeDtypeStruct((n_tok, D_MODEL), jnp.float32),
        scratch_types=[
            pltpu.VMEM((bpw,), jnp.int32),
            pltpu.VMEM((chunk, D_MODEL), jnp.float32),
            pltpu.VMEM((chunk, D_MODEL), jnp.float32),
            pltpu.SemaphoreType.DMA,
        ],
    )
    def sc_kernel(x_hbm, pe_hbm, tab_hbm, out_hbm, idx_v, rows_v, pe_v, sem):
        wid = lax.axis_index("s") * nc + lax.axis_index("c")
        base = wid * bpw
        pos_base = base % seq_len
        pltpu.sync_copy(x_hbm.at[pl.ds(base, bpw)], idx_v)

        @pl.loop(0, n_chunks)
        def _chunk(i):
            off = i * chunk
            gather = pltpu.async_copy(
                tab_hbm.at[idx_v.at[pl.ds(off, chunk)]], rows_v, sem
            )
            pltpu.sync_copy(pe_hbm.at[pl.ds(pos_base + off, chunk)], pe_v)
            gather.wait()

            @pl.loop(0, chunk)
            def _row(r):
                for k in range(vecs_per_row):
                    sl = pl.ds(k * LANES, LANES)
                    rows_v[r, sl] = rows_v[r, sl] * SCALE + pe_v[r, sl]

            pltpu.sync_copy(rows_v, out_hbm.at[pl.ds(base + off, chunk)])

    return sc_kernel(x_flat, pe, emb_table)


def kernel(x, emb_table):
    batch, seq_len = x.shape
    out = _embed(
        x.reshape(-1).astype(jnp.int32),
        _positional_table(),
        emb_table,
        batch=batch,
        seq_len=seq_len,
    )
    return out.reshape(batch, seq_len, D_MODEL)
